# Initial kernel scaffold; baseline (speedup 1.0000x reference)
import jax
import jax.numpy as jnp
from jax.experimental import pallas as pl


def kernel(x, W_enc, b_enc, W_dec, b_dec):
    raise NotImplementedError("write your pallas kernel here")


# speedup vs baseline: 1.3852x; 1.3852x over previous
import functools

import jax
import jax.numpy as jnp
from jax.experimental import pallas as pl
from jax.experimental.pallas import tpu as pltpu

K_TOP = 32


def _enc_body(x_ref, w_ref, b_ref, out_ref):
    acc = jax.lax.dot_general(
        x_ref[...], w_ref[...], (((1,), (1,)), ((), ())),
        preferred_element_type=jnp.float32)
    out_ref[...] = jnp.maximum(acc + b_ref[0, :][None, :], 0.0)


def _topk_body(pre_ref, h_ref, work_ref, *, B, H, k):
    work_ref[...] = pre_ref[...]
    col = jax.lax.broadcasted_iota(jnp.int32, (B, H), 1)

    def step(_, carry):
        w = work_ref[...]
        m = jnp.max(w, axis=1, keepdims=True)
        idx = jnp.min(jnp.where(w == m, col, H), axis=1, keepdims=True)
        work_ref[...] = jnp.where(col == idx, -jnp.inf, w)
        return carry

    jax.lax.fori_loop(0, k, step, 0)
    h_ref[...] = jnp.where(work_ref[...] == -jnp.inf, pre_ref[...], 0.0)


def _dec_body(h_ref, w_ref, b_ref, out_ref, *, B, D):
    @pl.when(pl.program_id(0) == 0)
    def _():
        out_ref[...] = jnp.broadcast_to(b_ref[0, :][None, :], (B, D))

    out_ref[...] += jax.lax.dot_general(
        h_ref[...], w_ref[...], (((1,), (1,)), ((), ())),
        preferred_element_type=jnp.float32)


def kernel(x, W_enc, b_enc, W_dec, b_dec):
    B, D = x.shape
    H = W_enc.shape[0]
    k = max(0, min(K_TOP, H))
    BH = 1024

    pre = pl.pallas_call(
        _enc_body,
        grid=(H // BH,),
        in_specs=[
            pl.BlockSpec((B, D), lambda j: (0, 0)),
            pl.BlockSpec((BH, D), lambda j: (j, 0)),
            pl.BlockSpec((1, BH), lambda j: (0, j)),
        ],
        out_specs=pl.BlockSpec((B, BH), lambda j: (0, j)),
        out_shape=jax.ShapeDtypeStruct((B, H), jnp.float32),
    )(x, W_enc, b_enc.reshape(1, H))

    h = pl.pallas_call(
        functools.partial(_topk_body, B=B, H=H, k=k),
        in_specs=[pl.BlockSpec((B, H), lambda: (0, 0))],
        out_specs=pl.BlockSpec((B, H), lambda: (0, 0)),
        out_shape=jax.ShapeDtypeStruct((B, H), jnp.float32),
        scratch_shapes=[pltpu.VMEM((B, H), jnp.float32)],
    )(pre)

    x_hat = pl.pallas_call(
        functools.partial(_dec_body, B=B, D=D),
        grid=(H // BH,),
        in_specs=[
            pl.BlockSpec((B, BH), lambda j: (0, j)),
            pl.BlockSpec((D, BH), lambda j: (0, j)),
            pl.BlockSpec((1, D), lambda j: (0, 0)),
        ],
        out_specs=pl.BlockSpec((B, D), lambda j: (0, 0)),
        out_shape=jax.ShapeDtypeStruct((B, D), jnp.float32),
    )(h, W_dec, b_dec.reshape(1, D))

    return (h, x_hat)


# speedup vs baseline: 1.3939x; 1.0063x over previous
import functools

import jax
import jax.numpy as jnp
from jax import lax
from jax.experimental import pallas as pl
from jax.experimental.pallas import tpu as pltpu
from jax.experimental.pallas import tpu_sc as plsc

K_TOP = 32
_L = 16
_CH = 16
_NC = 2
_NS = 16


def _enc_body(x_ref, w_ref, b_ref, out_ref):
    acc = jax.lax.dot_general(
        x_ref[...], w_ref[...], (((1,), (1,)), ((), ())),
        preferred_element_type=jnp.float32)
    out_ref[...] = jnp.maximum(acc + b_ref[0, :][None, :], 0.0)


def _topk_body(pre_ref, h_ref, work_ref, *, B, H, k):
    work_ref[...] = pre_ref[...]
    col = jax.lax.broadcasted_iota(jnp.int32, (B, H), 1)

    def step(_, carry):
        w = work_ref[...]
        m = jnp.max(w, axis=1, keepdims=True)
        idx = jnp.min(jnp.where(w == m, col, H), axis=1, keepdims=True)
        work_ref[...] = jnp.where(col == idx, -jnp.inf, w)
        return carry

    jax.lax.fori_loop(0, k, step, 0)
    h_ref[...] = jnp.where(work_ref[...] == -jnp.inf, pre_ref[...], 0.0)


def _topk_sc_body(pre_hbm, h_hbm, row_v, h_v, chunkv, chunki, sem, *, H, k,
                  num_cores):
    wid = lax.axis_index("s") * num_cores + lax.axis_index("c")
    cp = pltpu.async_copy(pre_hbm.at[wid], row_v, sem)

    zeros16 = jnp.zeros((_L,), jnp.float32)

    def zbody(j, c):
        h_v[pl.ds(j * _L, _L)] = zeros16
        return c

    lax.fori_loop(0, H // _L, zbody, 0)
    cp.wait()

    iota16 = lax.iota(jnp.int32, _L)
    csz = H // _CH
    nv = csz // _L

    for c in range(_CH):
        base = c * csz

        def p1(j, carry, base=base):
            bv, bi = carry
            off = base + j * _L
            v = row_v[pl.ds(off, _L)]
            take = v > bv
            return (jnp.where(take, v, bv),
                    jnp.where(take, off + iota16, bi))

        bv, bi = lax.fori_loop(
            1, nv, p1, (row_v[pl.ds(base, _L)], base + iota16))
        chunkv[pl.ds(c * _L, _L)] = bv
        chunki[pl.ds(c * _L, _L)] = bi

    big = jnp.int32(1 << 30)
    lane0 = iota16 == 0

    def _perm(v, perm):
        return v.at[perm].get(mode="promise_in_bounds")

    def _bfly(v, op):
        for s in (8, 4, 2, 1):
            v = op(v, _perm(v, iota16 ^ s))
        return v

    def emit(t, carry):
        bv = chunkv[pl.ds(0, _L)]
        bi = chunki[pl.ds(0, _L)]
        for c in range(1, _CH):
            v = chunkv[pl.ds(c * _L, _L)]
            i = chunki[pl.ds(c * _L, _L)]
            take = (v > bv) | ((v == bv) & (i < bi))
            bv = jnp.where(take, v, bv)
            bi = jnp.where(take, i, bi)
        m = _bfly(bv, jnp.maximum)
        idx = _bfly(jnp.where(bv == m, bi, big), jnp.minimum)
        plsc.store_scatter(h_v, [idx], m, mask=lane0)
        plsc.store_scatter(row_v, [idx], jnp.full((_L,), -1.0, jnp.float32),
                           mask=lane0)
        base = (idx // csz) * csz + idx % _L
        gv = gi = None
        for g in range(nv // _L):
            ii = base + _L * (iota16 + _L * g)
            vv = plsc.load_gather(row_v, [ii])
            if gv is None:
                gv, gi = vv, ii
            else:
                take = (vv > gv) | ((vv == gv) & (ii < gi))
                gv = jnp.where(take, vv, gv)
                gi = jnp.where(take, ii, gi)
        m2 = _bfly(gv, jnp.maximum)
        i2 = _bfly(jnp.where(gv == m2, gi, big), jnp.minimum)
        pos = (idx // csz) * _L + idx % _L
        plsc.store_scatter(chunkv, [pos], m2, mask=lane0)
        plsc.store_scatter(chunki, [pos], i2, mask=lane0)
        return carry

    lax.fori_loop(0, k, emit, 0)
    pltpu.sync_copy(h_v, h_hbm.at[wid])


def _topk_sc(pre, *, B, H, k):
    mesh = plsc.VectorSubcoreMesh(core_axis_name="c", subcore_axis_name="s",
                                  num_cores=_NC, num_subcores=_NS)
    return pl.kernel(
        functools.partial(_topk_sc_body, H=H, k=k, num_cores=_NC),
        out_type=jax.ShapeDtypeStruct((B, H), jnp.float32),
        mesh=mesh,
        compiler_params=pltpu.CompilerParams(needs_layout_passes=False),
        scratch_types=[
            pltpu.VMEM((H,), jnp.float32),
            pltpu.VMEM((H,), jnp.float32),
            pltpu.VMEM((_CH * _L,), jnp.float32),
            pltpu.VMEM((_CH * _L,), jnp.int32),
            pltpu.SemaphoreType.DMA,
        ],
    )(pre)


def _dec_body(h_ref, w_ref, b_ref, out_ref, *, B, D):
    @pl.when(pl.program_id(0) == 0)
    def _():
        out_ref[...] = jnp.broadcast_to(b_ref[0, :][None, :], (B, D))

    out_ref[...] += jax.lax.dot_general(
        h_ref[...], w_ref[...], (((1,), (1,)), ((), ())),
        preferred_element_type=jnp.float32)


def kernel(x, W_enc, b_enc, W_dec, b_dec):
    B, D = x.shape
    H = W_enc.shape[0]
    k = max(0, min(K_TOP, H))
    BH = 1024

    pre = pl.pallas_call(
        _enc_body,
        grid=(H // BH,),
        in_specs=[
            pl.BlockSpec((B, D), lambda j: (0, 0)),
            pl.BlockSpec((BH, D), lambda j: (j, 0)),
            pl.BlockSpec((1, BH), lambda j: (0, j)),
        ],
        out_specs=pl.BlockSpec((B, BH), lambda j: (0, j)),
        out_shape=jax.ShapeDtypeStruct((B, H), jnp.float32),
    )(x, W_enc, b_enc.reshape(1, H))

    h = _topk_sc(pre, B=B, H=H, k=k)

    x_hat = pl.pallas_call(
        functools.partial(_dec_body, B=B, D=D),
        grid=(H // BH,),
        in_specs=[
            pl.BlockSpec((B, BH), lambda j: (0, j)),
            pl.BlockSpec((D, BH), lambda j: (0, j)),
            pl.BlockSpec((1, D), lambda j: (0, 0)),
        ],
        out_specs=pl.BlockSpec((B, D), lambda j: (0, 0)),
        out_shape=jax.ShapeDtypeStruct((B, D), jnp.float32),
    )(h, W_dec, b_dec.reshape(1, D))

    return (h, x_hat)
